# baseline (speedup 1.0000x reference)
import jax
import jax.numpy as jnp
from jax import lax
from jax.experimental import pallas as pl
from jax.experimental.pallas import tpu as pltpu, tpu_sc as plsc

VOCAB = 1000000
EMB = 64
SEQ = 200
BATCH = 1024
B = BATCH * SEQ
NC, NS = 2, 16
NW = NC * NS
C = SEQ
N_CHUNKS = B // (NW * C)
NBUF = 4
LANES = 16
WPAD = 128


def _body(ids_hbm, word_hbm, pos_hbm, out_hbm,
          pos_v, idx_v, r0, r1, r2, r3,
          sg0, sg1, sg2, sg3, ss0, ss1, ss2, ss3):
    rows = (r0, r1, r2, r3)
    sg = (sg0, sg1, sg2, sg3)
    ss = (ss0, ss1, ss2, ss3)

    wid = lax.axis_index("s") * NC + lax.axis_index("c")
    base = wid * N_CHUNKS * C
    row0 = wid * N_CHUNKS

    pltpu.sync_copy(pos_hbm, pos_v)
    pltpu.sync_copy(ids_hbm.at[pl.ds(row0, N_CHUNKS)], idx_v)

    def gather_start(k, b):
        pltpu.make_async_copy(word_hbm.at[idx_v.at[k]], rows[b], sg[b]).start()

    def gather_wait(b):
        pltpu.make_async_copy(word_hbm.at[idx_v.at[0]], rows[b], sg[b]).wait()

    def store_start(k, b):
        pltpu.make_async_copy(rows[b].at[:, pl.ds(0, EMB)],
                              out_hbm.at[pl.ds(base + k * C, C)],
                              ss[b]).start()

    def store_wait(b):
        pltpu.make_async_copy(rows[b].at[:, pl.ds(0, EMB)],
                              out_hbm.at[pl.ds(base, C)],
                              ss[b]).wait()

    def add_pos(b):
        rb = rows[b]

        @plsc.parallel_loop(0, C, 1, unroll=8)
        def _(r):
            for o in range(EMB // LANES):
                sl = pl.ds(o * LANES, LANES)
                rb[r, sl] = rb[r, sl] + pos_v[r, sl]

    def chunk_body(k, b, *, wait_prev_store, next_k):
        if wait_prev_store:
            store_wait((b + 3) % NBUF)
        if next_k is not None:
            gather_start(next_k, (b + 3) % NBUF)
        gather_wait(b)
        add_pos(b)
        store_start(k, b)

    for b in range(NBUF - 1):
        gather_start(b, b)
    chunk_body(0, 0, wait_prev_store=False, next_k=3)
    for k in range(1, NBUF):
        chunk_body(k, k % NBUF, wait_prev_store=True, next_k=k + 3)

    def outer(g, _):
        for b in range(NBUF):
            k = NBUF * g + b
            chunk_body(k, b, wait_prev_store=True, next_k=k + 3)
        return ()

    lax.fori_loop(1, N_CHUNKS // NBUF - 1, outer, ())

    chunk_body(N_CHUNKS - 4, 0, wait_prev_store=True, next_k=N_CHUNKS - 1)
    for k in range(N_CHUNKS - 3, N_CHUNKS):
        chunk_body(k, k % NBUF, wait_prev_store=True, next_k=None)
    store_wait(3)


TCV = 2048


def _tc_tr_body(in_ref, out_ref):
    x = in_ref[...]
    r = lax.broadcasted_iota(jnp.int32, (EMB, EMB), 0)
    c = lax.broadcasted_iota(jnp.int32, (EMB, EMB), 1)
    eye = jnp.where(r == c, 1.0, 0.0).astype(jnp.float32)
    out_ref[:, 0:EMB] = lax.dot_general(
        x, eye, (((0,), (0,)), ((), ())),
        preferred_element_type=jnp.float32)


@jax.jit
def _tc_transpose(wordT):
    call = pl.pallas_call(
        _tc_tr_body,
        grid=(-(-VOCAB // TCV),),
        in_specs=[pl.BlockSpec((EMB, TCV), lambda k: (0, k))],
        out_specs=pl.BlockSpec((TCV, WPAD), lambda k: (k, 0)),
        out_shape=jax.ShapeDtypeStruct((VOCAB, WPAD), jnp.float32),
    )
    return call(wordT)


@jax.jit
def _embed(ids, word_pad, pos_table):
    kern = pl.kernel(
        _body,
        out_type=jax.ShapeDtypeStruct((B, EMB), jnp.float32),
        mesh=plsc.VectorSubcoreMesh(core_axis_name="c", subcore_axis_name="s"),
        scratch_types=[
            pltpu.VMEM((C, EMB), jnp.float32),
            pltpu.VMEM((N_CHUNKS, C), jnp.int32),
            pltpu.VMEM((C, WPAD), jnp.float32),
            pltpu.VMEM((C, WPAD), jnp.float32),
            pltpu.VMEM((C, WPAD), jnp.float32),
            pltpu.VMEM((C, WPAD), jnp.float32),
            pltpu.SemaphoreType.DMA,
            pltpu.SemaphoreType.DMA,
            pltpu.SemaphoreType.DMA,
            pltpu.SemaphoreType.DMA,
            pltpu.SemaphoreType.DMA,
            pltpu.SemaphoreType.DMA,
            pltpu.SemaphoreType.DMA,
            pltpu.SemaphoreType.DMA,
        ],
        compiler_params=pltpu.CompilerParams(use_tc_tiling_on_sc=False),
    )
    return kern(ids, word_pad, pos_table)


def kernel(input_ids, word_table, pos_table):
    ids = input_ids.astype(jnp.int32)
    word_pad = _tc_transpose(word_table.T)
    out = _embed(ids, word_pad, pos_table)
    return out.reshape(BATCH, SEQ, EMB)


# speedup vs baseline: 1.4579x; 1.4579x over previous
import jax
import jax.numpy as jnp
from jax import lax
from jax.experimental import pallas as pl
from jax.experimental.pallas import tpu as pltpu, tpu_sc as plsc

VOCAB = 1000000
EMB = 64
SEQ = 200
BATCH = 1024
B = BATCH * SEQ
NC, NS = 2, 16
NW = NC * NS
C = SEQ
N_CHUNKS = B // (NW * C)
NBUF = 4
LANES = 16
WPAD = 128


def _body(ids_hbm, word_hbm, pos_hbm, out_hbm,
          pos_v, idx_v, r0, r1, r2, r3,
          sg0, sg1, sg2, sg3, ss0, ss1, ss2, ss3):
    rows = (r0, r1, r2, r3)
    sg = (sg0, sg1, sg2, sg3)
    ss = (ss0, ss1, ss2, ss3)

    wid = lax.axis_index("s") * NC + lax.axis_index("c")
    base = wid * N_CHUNKS * C
    row0 = wid * N_CHUNKS

    pltpu.sync_copy(pos_hbm, pos_v)
    pltpu.sync_copy(ids_hbm.at[pl.ds(row0, N_CHUNKS)], idx_v)

    def gather_start(k, b):
        pltpu.make_async_copy(word_hbm.at[idx_v.at[k]], rows[b], sg[b]).start()

    def gather_wait(b):
        pltpu.make_async_copy(word_hbm.at[idx_v.at[0]], rows[b], sg[b]).wait()

    def store_start(k, b):
        pltpu.make_async_copy(rows[b].at[:, pl.ds(0, EMB)],
                              out_hbm.at[pl.ds(base + k * C, C)],
                              ss[b]).start()

    def store_wait(b):
        pltpu.make_async_copy(rows[b].at[:, pl.ds(0, EMB)],
                              out_hbm.at[pl.ds(base, C)],
                              ss[b]).wait()

    def add_pos(b):
        rb = rows[b]

        @plsc.parallel_loop(0, C, 1, unroll=8)
        def _(r):
            for o in range(EMB // LANES):
                sl = pl.ds(o * LANES, LANES)
                rb[r, sl] = rb[r, sl] + pos_v[r, sl]

    def chunk_body(k, b, *, wait_prev_store, next_k):
        if wait_prev_store:
            store_wait((b + 3) % NBUF)
        if next_k is not None:
            gather_start(next_k, (b + 3) % NBUF)
        gather_wait(b)
        add_pos(b)
        store_start(k, b)

    for b in range(NBUF - 1):
        gather_start(b, b)
    chunk_body(0, 0, wait_prev_store=False, next_k=3)
    for k in range(1, NBUF):
        chunk_body(k, k % NBUF, wait_prev_store=True, next_k=k + 3)

    def outer(g, _):
        for b in range(NBUF):
            k = NBUF * g + b
            chunk_body(k, b, wait_prev_store=True, next_k=k + 3)
        return ()

    lax.fori_loop(1, N_CHUNKS // NBUF - 1, outer, ())

    chunk_body(N_CHUNKS - 4, 0, wait_prev_store=True, next_k=N_CHUNKS - 1)
    for k in range(N_CHUNKS - 3, N_CHUNKS):
        chunk_body(k, k % NBUF, wait_prev_store=True, next_k=None)
    store_wait(3)


TCV = 8192


def _tc_tr_body(in_ref, out_ref):
    out_ref[:, 0:EMB] = in_ref[...].T


@jax.jit
def _tc_transpose(wordT):
    call = pl.pallas_call(
        _tc_tr_body,
        grid=(-(-VOCAB // TCV),),
        in_specs=[pl.BlockSpec((EMB, TCV), lambda k: (0, k))],
        out_specs=pl.BlockSpec((TCV, WPAD), lambda k: (k, 0)),
        out_shape=jax.ShapeDtypeStruct((VOCAB, WPAD), jnp.float32),
    )
    return call(wordT)


@jax.jit
def _embed(ids, word_pad, pos_table):
    kern = pl.kernel(
        _body,
        out_type=jax.ShapeDtypeStruct((B, EMB), jnp.float32),
        mesh=plsc.VectorSubcoreMesh(core_axis_name="c", subcore_axis_name="s"),
        scratch_types=[
            pltpu.VMEM((C, EMB), jnp.float32),
            pltpu.VMEM((N_CHUNKS, C), jnp.int32),
            pltpu.VMEM((C, WPAD), jnp.float32),
            pltpu.VMEM((C, WPAD), jnp.float32),
            pltpu.VMEM((C, WPAD), jnp.float32),
            pltpu.VMEM((C, WPAD), jnp.float32),
            pltpu.SemaphoreType.DMA,
            pltpu.SemaphoreType.DMA,
            pltpu.SemaphoreType.DMA,
            pltpu.SemaphoreType.DMA,
            pltpu.SemaphoreType.DMA,
            pltpu.SemaphoreType.DMA,
            pltpu.SemaphoreType.DMA,
            pltpu.SemaphoreType.DMA,
        ],
        compiler_params=pltpu.CompilerParams(use_tc_tiling_on_sc=False),
    )
    return kern(ids, word_pad, pos_table)


def kernel(input_ids, word_table, pos_table):
    ids = input_ids.astype(jnp.int32)
    word_pad = _tc_transpose(word_table.T)
    out = _embed(ids, word_pad, pos_table)
    return out.reshape(BATCH, SEQ, EMB)


# speedup vs baseline: 1.5154x; 1.0395x over previous
import jax
import jax.numpy as jnp
from jax import lax
from jax.experimental import pallas as pl
from jax.experimental.pallas import tpu as pltpu, tpu_sc as plsc

VOCAB = 1000000
EMB = 64
SEQ = 200
BATCH = 1024
B = BATCH * SEQ
NC, NS = 2, 16
NW = NC * NS
C = SEQ
N_CHUNKS = B // (NW * C)
NBUF = 4
LANES = 16
WPAD = 128


def _body(ids_hbm, word_hbm, pos_hbm, out_hbm,
          pos_v, idx_v, r0, r1, r2, r3,
          sg0, sg1, sg2, sg3, ss0, ss1, ss2, ss3):
    rows = (r0, r1, r2, r3)
    sg = (sg0, sg1, sg2, sg3)
    ss = (ss0, ss1, ss2, ss3)

    wid = lax.axis_index("s") * NC + lax.axis_index("c")
    base = wid * N_CHUNKS * C
    row0 = wid * N_CHUNKS

    pltpu.sync_copy(pos_hbm, pos_v)
    pltpu.sync_copy(ids_hbm.at[pl.ds(row0, N_CHUNKS)], idx_v)

    def gather_start(k, b):
        pltpu.make_async_copy(word_hbm.at[idx_v.at[k]], rows[b], sg[b]).start()

    def gather_wait(b):
        pltpu.make_async_copy(word_hbm.at[idx_v.at[0]], rows[b], sg[b]).wait()

    def store_start(k, b):
        pltpu.make_async_copy(rows[b].at[:, pl.ds(0, EMB)],
                              out_hbm.at[pl.ds(base + k * C, C)],
                              ss[b]).start()

    def store_wait(b):
        pltpu.make_async_copy(rows[b].at[:, pl.ds(0, EMB)],
                              out_hbm.at[pl.ds(base, C)],
                              ss[b]).wait()

    def add_pos(b):
        rb = rows[b]

        @plsc.parallel_loop(0, C, 1, unroll=8)
        def _(r):
            for o in range(EMB // LANES):
                sl = pl.ds(o * LANES, LANES)
                rb[r, sl] = rb[r, sl] + pos_v[r, sl]

    def chunk_body(k, b, *, wait_prev_store, next_k):
        if wait_prev_store:
            store_wait((b + 3) % NBUF)
        if next_k is not None:
            gather_start(next_k, (b + 3) % NBUF)
        gather_wait(b)
        add_pos(b)
        store_start(k, b)

    for b in range(NBUF - 1):
        gather_start(b, b)
    chunk_body(0, 0, wait_prev_store=False, next_k=3)
    for k in range(1, NBUF):
        chunk_body(k, k % NBUF, wait_prev_store=True, next_k=k + 3)

    def outer(g, _):
        for b in range(NBUF):
            k = NBUF * g + b
            chunk_body(k, b, wait_prev_store=True, next_k=k + 3)
        return ()

    lax.fori_loop(1, N_CHUNKS // NBUF - 1, outer, ())

    chunk_body(N_CHUNKS - 4, 0, wait_prev_store=True, next_k=N_CHUNKS - 1)
    for k in range(N_CHUNKS - 3, N_CHUNKS):
        chunk_body(k, k % NBUF, wait_prev_store=True, next_k=None)
    store_wait(3)


TCV = 16384


def _tc_tr_body(in_ref, out_ref):
    out_ref[:, 0:EMB] = in_ref[...].T


@jax.jit
def _tc_transpose(wordT):
    call = pl.pallas_call(
        _tc_tr_body,
        grid=(-(-VOCAB // TCV),),
        in_specs=[pl.BlockSpec((EMB, TCV), lambda k: (0, k))],
        out_specs=pl.BlockSpec((TCV, WPAD), lambda k: (k, 0)),
        out_shape=jax.ShapeDtypeStruct((VOCAB, WPAD), jnp.float32),
    )
    return call(wordT)


@jax.jit
def _embed(ids, word_pad, pos_table):
    kern = pl.kernel(
        _body,
        out_type=jax.ShapeDtypeStruct((B, EMB), jnp.float32),
        mesh=plsc.VectorSubcoreMesh(core_axis_name="c", subcore_axis_name="s"),
        scratch_types=[
            pltpu.VMEM((C, EMB), jnp.float32),
            pltpu.VMEM((N_CHUNKS, C), jnp.int32),
            pltpu.VMEM((C, WPAD), jnp.float32),
            pltpu.VMEM((C, WPAD), jnp.float32),
            pltpu.VMEM((C, WPAD), jnp.float32),
            pltpu.VMEM((C, WPAD), jnp.float32),
            pltpu.SemaphoreType.DMA,
            pltpu.SemaphoreType.DMA,
            pltpu.SemaphoreType.DMA,
            pltpu.SemaphoreType.DMA,
            pltpu.SemaphoreType.DMA,
            pltpu.SemaphoreType.DMA,
            pltpu.SemaphoreType.DMA,
            pltpu.SemaphoreType.DMA,
        ],
        compiler_params=pltpu.CompilerParams(use_tc_tiling_on_sc=False),
    )
    return kern(ids, word_pad, pos_table)


def kernel(input_ids, word_table, pos_table):
    ids = input_ids.astype(jnp.int32)
    word_pad = _tc_transpose(word_table.T)
    out = _embed(ids, word_pad, pos_table)
    return out.reshape(BATCH, SEQ, EMB)


# speedup vs baseline: 1.5369x; 1.0142x over previous
import jax
import jax.numpy as jnp
from jax import lax
from jax.experimental import pallas as pl
from jax.experimental.pallas import tpu as pltpu, tpu_sc as plsc

VOCAB = 1000000
EMB = 64
SEQ = 200
BATCH = 1024
B = BATCH * SEQ
NC, NS = 2, 16
NW = NC * NS
C = SEQ
N_CHUNKS = B // (NW * C)
NBUF = 4
LANES = 16
WPAD = 128


def _body(ids_hbm, word_hbm, pos_hbm, out_hbm,
          pos_v, idx_v, r0, r1, r2, r3,
          sg0, sg1, sg2, sg3, ss0, ss1, ss2, ss3):
    rows = (r0, r1, r2, r3)
    sg = (sg0, sg1, sg2, sg3)
    ss = (ss0, ss1, ss2, ss3)

    wid = lax.axis_index("s") * NC + lax.axis_index("c")
    base = wid * N_CHUNKS * C
    row0 = wid * N_CHUNKS

    pltpu.sync_copy(pos_hbm, pos_v)
    pltpu.sync_copy(ids_hbm.at[pl.ds(row0, N_CHUNKS)], idx_v)

    def gather_start(k, b):
        pltpu.make_async_copy(word_hbm.at[idx_v.at[k]], rows[b], sg[b]).start()

    def gather_wait(b):
        pltpu.make_async_copy(word_hbm.at[idx_v.at[0]], rows[b], sg[b]).wait()

    def store_start(k, b):
        pltpu.make_async_copy(rows[b].at[:, pl.ds(0, EMB)],
                              out_hbm.at[pl.ds(base + k * C, C)],
                              ss[b]).start()

    def store_wait(b):
        pltpu.make_async_copy(rows[b].at[:, pl.ds(0, EMB)],
                              out_hbm.at[pl.ds(base, C)],
                              ss[b]).wait()

    def add_pos(b):
        rb = rows[b]

        @plsc.parallel_loop(0, C, 1, unroll=8)
        def _(r):
            for o in range(EMB // LANES):
                sl = pl.ds(o * LANES, LANES)
                rb[r, sl] = rb[r, sl] + pos_v[r, sl]

    def chunk_body(k, b, *, wait_prev_store, next_k):
        if wait_prev_store:
            store_wait((b + 3) % NBUF)
        if next_k is not None:
            gather_start(next_k, (b + 3) % NBUF)
        gather_wait(b)
        add_pos(b)
        store_start(k, b)

    for b in range(NBUF - 1):
        gather_start(b, b)
    chunk_body(0, 0, wait_prev_store=False, next_k=3)
    for k in range(1, NBUF):
        chunk_body(k, k % NBUF, wait_prev_store=True, next_k=k + 3)

    def outer(g, _):
        for b in range(NBUF):
            k = NBUF * g + b
            chunk_body(k, b, wait_prev_store=True, next_k=k + 3)
        return ()

    lax.fori_loop(1, N_CHUNKS // NBUF - 1, outer, ())

    chunk_body(N_CHUNKS - 4, 0, wait_prev_store=True, next_k=N_CHUNKS - 1)
    for k in range(N_CHUNKS - 3, N_CHUNKS):
        chunk_body(k, k % NBUF, wait_prev_store=True, next_k=None)
    store_wait(3)


TCV = 32768


def _tc_tr_body(in_ref, out_ref):
    out_ref[:, 0:EMB] = in_ref[...].T


@jax.jit
def _tc_transpose(wordT):
    call = pl.pallas_call(
        _tc_tr_body,
        grid=(-(-VOCAB // TCV),),
        in_specs=[pl.BlockSpec((EMB, TCV), lambda k: (0, k))],
        out_specs=pl.BlockSpec((TCV, WPAD), lambda k: (k, 0)),
        out_shape=jax.ShapeDtypeStruct((VOCAB, WPAD), jnp.float32),
    )
    return call(wordT)


@jax.jit
def _embed(ids, word_pad, pos_table):
    kern = pl.kernel(
        _body,
        out_type=jax.ShapeDtypeStruct((B, EMB), jnp.float32),
        mesh=plsc.VectorSubcoreMesh(core_axis_name="c", subcore_axis_name="s"),
        scratch_types=[
            pltpu.VMEM((C, EMB), jnp.float32),
            pltpu.VMEM((N_CHUNKS, C), jnp.int32),
            pltpu.VMEM((C, WPAD), jnp.float32),
            pltpu.VMEM((C, WPAD), jnp.float32),
            pltpu.VMEM((C, WPAD), jnp.float32),
            pltpu.VMEM((C, WPAD), jnp.float32),
            pltpu.SemaphoreType.DMA,
            pltpu.SemaphoreType.DMA,
            pltpu.SemaphoreType.DMA,
            pltpu.SemaphoreType.DMA,
            pltpu.SemaphoreType.DMA,
            pltpu.SemaphoreType.DMA,
            pltpu.SemaphoreType.DMA,
            pltpu.SemaphoreType.DMA,
        ],
        compiler_params=pltpu.CompilerParams(use_tc_tiling_on_sc=False),
    )
    return kern(ids, word_pad, pos_table)


def kernel(input_ids, word_table, pos_table):
    ids = input_ids.astype(jnp.int32)
    word_pad = _tc_transpose(word_table.T)
    out = _embed(ids, word_pad, pos_table)
    return out.reshape(BATCH, SEQ, EMB)
